# baseline (speedup 1.0000x reference)
import functools

import jax
import jax.numpy as jnp
from jax import lax
from jax.experimental import pallas as pl
from jax.experimental.pallas import tpu as pltpu
from jax.experimental.pallas import tpu_sc as plsc

N_NODES = 100000
N_EDGES = 1600000
NUM_GRAPHS = 64
EPS = 1e-7

NPAD = 100352
NCORES = 2
NTILES = 16
EPT = N_EDGES // NTILES
EB = 2000
NBLK = EPT // EB
SUB = 128


def _make_agg(d, nchunk_local, nstatic):
    mesh = plsc.VectorSubcoreMesh(core_axis_name="c", subcore_axis_name="s")
    chunk = NPAD // (NCORES * nchunk_local)
    acc_rows = chunk + 128
    wb = chunk // NTILES

    @functools.partial(
        pl.kernel,
        out_type=jax.ShapeDtypeStruct((NPAD, d), jnp.float32),
        mesh=mesh,
        scratch_types=[
            pltpu.VMEM((2, EB), jnp.int32),
            pltpu.VMEM((2, EB), jnp.int32),
            pltpu.VMEM((EB + 176,), jnp.int32),
            pltpu.VMEM((EB + 176,), jnp.int32),
            pltpu.VMEM((nstatic, SUB), jnp.int32),
            pltpu.VMEM((nstatic * SUB, d), jnp.float32),
            pltpu.VMEM((64, d), jnp.float32),
            pltpu.VMEM_SHARED((acc_rows, d), jnp.float32),
            pltpu.SemaphoreType.DMA,
            pltpu.SemaphoreType.DMA,
            pltpu.SemaphoreType.DMA,
            pltpu.SemaphoreType.DMA,
        ],
        compiler_params=pltpu.CompilerParams(use_tc_tiling_on_sc=False,
                                             needs_layout_passes=False),
    )
    def agg_kernel(g_hbm, edges_hbm, agg_hbm, sblk, dblk, csrc, cdst, stage,
                   rows, zbuf, acc, sem_e0, sem_e1, sem_g, sem_s):
        cid = lax.axis_index("c")
        tid = lax.axis_index("s")
        ebase = tid * EPT
        sem_e = (sem_e0, sem_e1)

        @pl.loop(0, 64)
        def _(i):
            for j in range(d // 16):
                zbuf[i, pl.ds(j * 16, 16)] = jnp.zeros((16,), jnp.float32)

        def issue_edges(slot, b):
            off = ebase + b * EB
            pltpu.async_copy(edges_hbm.at[0, pl.ds(off, EB)], sblk.at[slot],
                             sem_e[slot])
            pltpu.async_copy(edges_hbm.at[1, pl.ds(off, EB)], dblk.at[slot],
                             sem_e[slot])

        def wait_edges(slot, b):
            off = ebase + b * EB
            pltpu.make_async_copy(edges_hbm.at[0, pl.ds(off, EB)],
                                  sblk.at[slot], sem_e[slot]).wait()
            pltpu.make_async_copy(edges_hbm.at[1, pl.ds(off, EB)],
                                  dblk.at[slot], sem_e[slot]).wait()

        def fire_dummy_scatters():
            for j in range(nstatic):
                for jj in range(SUB // 16):
                    stage[j, pl.ds(jj * 16, 16)] = jnp.full((16,), chunk,
                                                            jnp.int32)
            for j in range(nstatic):
                pltpu.async_copy(rows.at[pl.ds(j * SUB, SUB)],
                                 acc.at[stage.at[j]], sem_s, add=True)

        def drain_scatters():
            for j in range(nstatic):
                pltpu.make_async_copy(rows.at[pl.ds(j * SUB, SUB)],
                                      acc.at[stage.at[j]], sem_s).wait()

        for half in range(nchunk_local):
            lo = (cid * nchunk_local + half) * chunk

            nslab = acc_rows // 64
            ntile_slabs = jnp.where(tid < nslab % NTILES,
                                    nslab // NTILES + 1, nslab // NTILES)

            def zero_body(i, _):
                r = tid + i * NTILES
                pltpu.sync_copy(zbuf, acc.at[pl.ds(r * 64, 64)])
                return _
            lax.fori_loop(0, ntile_slabs, zero_body, 0)
            plsc.subcore_barrier()

            def process(slot, b):
                sb_ref = sblk.at[slot]
                db_ref = dblk.at[slot]

                def scan_body(i, cnt):
                    sv = sb_ref[pl.ds(i * 16, 16)]
                    dv = db_ref[pl.ds(i * 16, 16)]
                    dl = dv - lo
                    m = (dl >= 0) & (dl < chunk)
                    plsc.store_compressed(csrc.at[pl.ds(cnt, 16)], sv, mask=m)
                    plsc.store_compressed(cdst.at[pl.ds(cnt, 16)], dl, mask=m)
                    return cnt + jnp.sum(m.astype(jnp.int32))

                cnt = lax.fori_loop(0, EB // 16, scan_body, jnp.int32(0))

                for j in range(SUB // 16):
                    csrc[pl.ds(cnt + j * 16, 16)] = jnp.zeros((16,), jnp.int32)
                    cdst[pl.ds(cnt + j * 16, 16)] = jnp.full((16,), chunk,
                                                             jnp.int32)
                nfill = (nstatic * SUB - cnt - SUB + 15) // 16

                def fill_body(k, car):
                    off = cnt + SUB + k * 16
                    csrc[pl.ds(off, 16)] = jnp.zeros((16,), jnp.int32)
                    cdst[pl.ds(off, 16)] = jnp.full((16,), chunk, jnp.int32)
                    return car
                lax.fori_loop(0, nfill, fill_body, 0)

                drain_scatters()
                for j in range(nstatic):
                    pltpu.async_copy(g_hbm.at[csrc.at[pl.ds(j * SUB, SUB)]],
                                     rows.at[pl.ds(j * SUB, SUB)], sem_g)
                for j in range(nstatic):
                    pltpu.make_async_copy(
                        g_hbm.at[csrc.at[pl.ds(j * SUB, SUB)]],
                        rows.at[pl.ds(j * SUB, SUB)], sem_g).wait()
                for j in range(nstatic):
                    for jj in range(SUB // 16):
                        stage[j, pl.ds(jj * 16, 16)] = (
                            cdst[pl.ds(j * SUB + jj * 16, 16)])
                for j in range(nstatic):
                    pltpu.async_copy(rows.at[pl.ds(j * SUB, SUB)],
                                     acc.at[stage.at[j]], sem_s, add=True)

                nsub = (cnt + SUB - 1) // SUB

                @pl.when(nsub > nstatic)
                def _():
                    drain_scatters()

                    def over_body(sb, car):
                        base = sb * SUB
                        for jj in range(SUB // 16):
                            stage[0, pl.ds(jj * 16, 16)] = (
                                cdst[pl.ds(base + jj * 16, 16)])
                        pltpu.sync_copy(g_hbm.at[csrc.at[pl.ds(base, SUB)]],
                                        rows.at[pl.ds(0, SUB)])
                        pltpu.sync_copy(rows.at[pl.ds(0, SUB)],
                                        acc.at[stage.at[0]], add=True)
                        return car
                    lax.fori_loop(nstatic, nsub, over_body, 0)
                    fire_dummy_scatters()

            fire_dummy_scatters()
            issue_edges(0, 0)
            issue_edges(1, 1)

            @pl.loop(0, NBLK // 2 - 1)
            def _(i):
                b0 = 2 * i
                b1 = 2 * i + 1
                wait_edges(0, b0)
                process(0, b0)
                issue_edges(0, b0 + 2)
                wait_edges(1, b1)
                process(1, b1)
                issue_edges(1, b1 + 2)

            wait_edges(0, NBLK - 2)
            process(0, NBLK - 2)
            wait_edges(1, NBLK - 1)
            process(1, NBLK - 1)
            drain_scatters()

            plsc.subcore_barrier()
            pltpu.sync_copy(acc.at[pl.ds(tid * wb, wb)],
                            agg_hbm.at[pl.ds(lo + tid * wb, wb)])
            plsc.subcore_barrier()

    return agg_kernel


def _make_emb_lookup(v, d):
    mesh = plsc.VectorSubcoreMesh(core_axis_name="c", subcore_axis_name="s")
    per_w = NPAD // (NCORES * NTILES)
    blk = 112
    nblk = per_w // blk

    @functools.partial(
        pl.kernel,
        out_type=jax.ShapeDtypeStruct((NPAD, d), jnp.float32),
        mesh=mesh,
        scratch_types=[
            pltpu.VMEM((blk,), jnp.int32),
            pltpu.VMEM((blk, d), jnp.float32),
        ],
        compiler_params=pltpu.CompilerParams(use_tc_tiling_on_sc=False),
    )
    def emb_kernel(emb_hbm, x_hbm, out_hbm, idx_v, rows_v):
        wid = lax.axis_index("s") * NCORES + lax.axis_index("c")
        base = wid * per_w

        @pl.loop(0, nblk)
        def _(b):
            off = base + b * blk
            pltpu.sync_copy(x_hbm.at[pl.ds(off, blk)], idx_v)
            pltpu.sync_copy(emb_hbm.at[idx_v], rows_v)
            pltpu.sync_copy(rows_v, out_hbm.at[pl.ds(off, blk)])

    return emb_kernel


ROWB = 2048


def _pre_kernel(h_ref, ws_ref, bs_ref, wd_ref, bd_ref, g_ref, hd_ref):
    h = h_ref[...]
    hs = jnp.dot(h, ws_ref[...], preferred_element_type=jnp.float32) + bs_ref[...]
    g_ref[...] = jnp.maximum(hs, 0.0) + EPS
    hd_ref[...] = (jnp.dot(h, wd_ref[...], preferred_element_type=jnp.float32)
                   + bd_ref[...])


def _tc_pre(h, ws, bs, wd, bd):
    din, dout = ws.shape
    grid = NPAD // ROWB
    return pl.pallas_call(
        _pre_kernel,
        grid=(grid,),
        in_specs=[
            pl.BlockSpec((ROWB, din), lambda i: (i, 0)),
            pl.BlockSpec((din, dout), lambda i: (0, 0)),
            pl.BlockSpec((1, dout), lambda i: (0, 0)),
            pl.BlockSpec((din, dout), lambda i: (0, 0)),
            pl.BlockSpec((1, dout), lambda i: (0, 0)),
        ],
        out_specs=[
            pl.BlockSpec((ROWB, dout), lambda i: (i, 0)),
            pl.BlockSpec((ROWB, dout), lambda i: (i, 0)),
        ],
        out_shape=[
            jax.ShapeDtypeStruct((NPAD, dout), jnp.float32),
            jax.ShapeDtypeStruct((NPAD, dout), jnp.float32),
        ],
    )(h, ws, bs.reshape(1, -1), wd, bd.reshape(1, -1))


def _post_kernel(make_g, agg_ref, hd_ref, w_ref, b_ref, *out_refs):
    s = agg_ref[...] + hd_ref[...]
    h = jnp.dot(s, w_ref[...], preferred_element_type=jnp.float32) + b_ref[...]
    out_refs[0][...] = h
    if make_g:
        out_refs[1][...] = jnp.maximum(h, 0.0) + EPS


def _tc_post(agg, hd, w, b, make_g):
    din, dout = w.shape
    grid = NPAD // ROWB
    out_specs = [pl.BlockSpec((ROWB, dout), lambda i: (i, 0))]
    out_shape = [jax.ShapeDtypeStruct((NPAD, dout), jnp.float32)]
    if make_g:
        out_specs.append(pl.BlockSpec((ROWB, dout), lambda i: (i, 0)))
        out_shape.append(jax.ShapeDtypeStruct((NPAD, dout), jnp.float32))
    return pl.pallas_call(
        functools.partial(_post_kernel, make_g),
        grid=(grid,),
        in_specs=[
            pl.BlockSpec((ROWB, din), lambda i: (i, 0)),
            pl.BlockSpec((ROWB, din), lambda i: (i, 0)),
            pl.BlockSpec((din, dout), lambda i: (0, 0)),
            pl.BlockSpec((1, dout), lambda i: (0, 0)),
        ],
        out_specs=out_specs,
        out_shape=out_shape,
    )(agg, hd, w, b.reshape(1, -1))


def _pool_kernel(h_ref, bf_ref, demo_ref, w1a_ref, w1b_ref, b1_ref, w2_ref,
                 b2_ref, out_ref, sums_ref, cnts_ref):
    i = pl.program_id(0)
    n = pl.num_programs(0)

    @pl.when(i == 0)
    def _():
        sums_ref[...] = jnp.zeros_like(sums_ref)
        cnts_ref[...] = jnp.zeros_like(cnts_ref)

    gids = lax.broadcasted_iota(jnp.int32, (1, NUM_GRAPHS), 1)
    onehot = (bf_ref[...] == gids).astype(jnp.float32)
    sums_ref[...] += lax.dot_general(
        onehot, h_ref[...], (((0,), (0,)), ((), ())),
        preferred_element_type=jnp.float32)
    cnts_ref[...] += lax.dot_general(
        onehot, jnp.ones((ROWB, 1), jnp.float32), (((0,), (0,)), ((), ())),
        preferred_element_type=jnp.float32)

    @pl.when(i == n - 1)
    def _():
        gf = sums_ref[...] / jnp.maximum(cnts_ref[...], 1.0)
        z = (jnp.dot(gf, w1a_ref[...], preferred_element_type=jnp.float32)
             + jnp.dot(demo_ref[...], w1b_ref[...],
                       preferred_element_type=jnp.float32)
             + b1_ref[...])
        z = jnp.maximum(z, 0.0)
        out_ref[...] = (jnp.dot(z, w2_ref[...],
                                preferred_element_type=jnp.float32)
                        + b2_ref[...])


def _tc_pool_cls(h, batch_f, demo, w1, b1, w2, b2):
    grid = NPAD // ROWB
    md = w1.shape[1]
    od = w2.shape[1]
    nd = demo.shape[1]
    return pl.pallas_call(
        _pool_kernel,
        grid=(grid,),
        in_specs=[
            pl.BlockSpec((ROWB, h.shape[1]), lambda i: (i, 0)),
            pl.BlockSpec((ROWB, 1), lambda i: (i, 0)),
            pl.BlockSpec((NUM_GRAPHS, nd), lambda i: (0, 0)),
            pl.BlockSpec((NUM_GRAPHS, md), lambda i: (0, 0)),
            pl.BlockSpec((nd, md), lambda i: (0, 0)),
            pl.BlockSpec((1, md), lambda i: (0, 0)),
            pl.BlockSpec((md, od), lambda i: (0, 0)),
            pl.BlockSpec((1, od), lambda i: (0, 0)),
        ],
        out_specs=pl.BlockSpec((NUM_GRAPHS, od), lambda i: (0, 0)),
        out_shape=jax.ShapeDtypeStruct((NUM_GRAPHS, od), jnp.float32),
        scratch_shapes=[
            pltpu.VMEM((NUM_GRAPHS, NUM_GRAPHS), jnp.float32),
            pltpu.VMEM((NUM_GRAPHS, 1), jnp.float32),
        ],
    )(h, batch_f, demo, w1[:NUM_GRAPHS], w1[NUM_GRAPHS:], b1.reshape(1, -1),
      w2, b2.reshape(1, -1))


def kernel(x, edge_index, batch, demographics, emb, l0_src_w, l0_src_b, l0_dst_w,
           l0_dst_b, l0_mlp_w, l0_mlp_b, l1_mlp_w, l1_mlp_b, l2_src_w, l2_src_b,
           l2_dst_w, l2_dst_b, l2_mlp_w, l2_mlp_b, cls_w1, cls_b1, cls_w2, cls_b2):
    pad = NPAD - N_NODES
    x_pad = jnp.concatenate([x.astype(jnp.int32), jnp.zeros((pad,), jnp.int32)])
    batch_f = jnp.concatenate(
        [batch.astype(jnp.int32),
         jnp.full((pad,), NUM_GRAPHS, jnp.int32)]).reshape(NPAD, 1)
    edges = edge_index.astype(jnp.int32)

    agg48 = _make_agg(48, 2, 5)
    agg64 = _make_agg(64, 4, 3)

    h = _make_emb_lookup(emb.shape[0], emb.shape[1])(emb, x_pad)
    g, hd = _tc_pre(h, l0_src_w, l0_src_b, l0_dst_w, l0_dst_b)
    agg = agg48(g, edges)
    h, g = _tc_post(agg, hd, l0_mlp_w, l0_mlp_b, make_g=True)
    agg = agg48(g, edges)
    (h,) = _tc_post(agg, h, l1_mlp_w, l1_mlp_b, make_g=False)
    g, hd = _tc_pre(h, l2_src_w, l2_src_b, l2_dst_w, l2_dst_b)
    agg = agg64(g, edges)
    (h,) = _tc_post(agg, hd, l2_mlp_w, l2_mlp_b, make_g=False)
    return _tc_pool_cls(h, batch_f, demographics, cls_w1, cls_b1, cls_w2, cls_b2)


# speedup vs baseline: 13.6566x; 13.6566x over previous
import functools

import jax
import jax.numpy as jnp
from jax import lax
from jax.experimental import pallas as pl
from jax.experimental.pallas import tpu as pltpu
from jax.experimental.pallas import tpu_sc as plsc

N_NODES = 100000
N_EDGES = 1600000
NUM_GRAPHS = 64
EPS = 1e-7

NPAD = 100352
NCORES = 2
NTILES = 16
EPT = N_EDGES // NTILES
EB = 2000
NBLK = EPT // EB
SUB = 128


def _make_agg(d, nchunk_local, nstatic):
    mesh = plsc.VectorSubcoreMesh(core_axis_name="c", subcore_axis_name="s")
    chunk = NPAD // (NCORES * nchunk_local)
    acc_rows = chunk + 128
    wb = chunk // NTILES

    @functools.partial(
        pl.kernel,
        out_type=jax.ShapeDtypeStruct((NPAD, d), jnp.float32),
        mesh=mesh,
        scratch_types=[
            pltpu.VMEM((2, EB), jnp.int32),
            pltpu.VMEM((2, EB), jnp.int32),
            pltpu.VMEM((EB + 176,), jnp.int32),
            pltpu.VMEM((EB + 176,), jnp.int32),
            pltpu.VMEM((nstatic, SUB), jnp.int32),
            pltpu.VMEM((nstatic * SUB, d), jnp.float32),
            pltpu.VMEM((64, d), jnp.float32),
            pltpu.VMEM_SHARED((acc_rows, d), jnp.float32),
            pltpu.SemaphoreType.DMA,
            pltpu.SemaphoreType.DMA,
            pltpu.SemaphoreType.DMA,
            pltpu.SemaphoreType.DMA,
        ],
        compiler_params=pltpu.CompilerParams(use_tc_tiling_on_sc=False,
                                             needs_layout_passes=False),
    )
    def agg_kernel(g_hbm, edges_hbm, agg_hbm, sblk, dblk, csrc, cdst, stage,
                   rows, zbuf, acc, sem_e0, sem_e1, sem_g, sem_s):
        cid = lax.axis_index("c")
        tid = lax.axis_index("s")
        ebase = tid * EPT
        sem_e = (sem_e0, sem_e1)

        @pl.loop(0, 64)
        def _(i):
            for j in range(d // 16):
                zbuf[i, pl.ds(j * 16, 16)] = jnp.zeros((16,), jnp.float32)

        def issue_edges(slot, b):
            off = ebase + b * EB
            pltpu.async_copy(edges_hbm.at[0, pl.ds(off, EB)], sblk.at[slot],
                             sem_e[slot])
            pltpu.async_copy(edges_hbm.at[1, pl.ds(off, EB)], dblk.at[slot],
                             sem_e[slot])

        def wait_edges(slot, b):
            off = ebase + b * EB
            pltpu.make_async_copy(edges_hbm.at[0, pl.ds(off, EB)],
                                  sblk.at[slot], sem_e[slot]).wait()
            pltpu.make_async_copy(edges_hbm.at[1, pl.ds(off, EB)],
                                  dblk.at[slot], sem_e[slot]).wait()

        def fire_dummy_scatters():
            for j in range(nstatic):
                for jj in range(SUB // 16):
                    stage[j, pl.ds(jj * 16, 16)] = (
                        chunk + jj * 16 + lax.iota(jnp.int32, 16))
            for j in range(nstatic):
                pltpu.async_copy(rows.at[pl.ds(j * SUB, SUB)],
                                 acc.at[stage.at[j]], sem_s, add=True)

        def drain_scatters():
            for j in range(nstatic):
                pltpu.make_async_copy(rows.at[pl.ds(j * SUB, SUB)],
                                      acc.at[stage.at[j]], sem_s).wait()

        for half in range(nchunk_local):
            lo = (cid * nchunk_local + half) * chunk

            nslab = acc_rows // 64
            ntile_slabs = jnp.where(tid < nslab % NTILES,
                                    nslab // NTILES + 1, nslab // NTILES)

            def zero_body(i, _):
                r = tid + i * NTILES
                pltpu.sync_copy(zbuf, acc.at[pl.ds(r * 64, 64)])
                return _
            lax.fori_loop(0, ntile_slabs, zero_body, 0)
            plsc.subcore_barrier()

            def process(slot, b):
                sb_ref = sblk.at[slot]
                db_ref = dblk.at[slot]

                def scan_body(i, cnt):
                    sv = sb_ref[pl.ds(i * 16, 16)]
                    dv = db_ref[pl.ds(i * 16, 16)]
                    dl = dv - lo
                    m = (dl >= 0) & (dl < chunk)
                    plsc.store_compressed(csrc.at[pl.ds(cnt, 16)], sv, mask=m)
                    plsc.store_compressed(cdst.at[pl.ds(cnt, 16)], dl, mask=m)
                    return cnt + jnp.sum(m.astype(jnp.int32))

                cnt = lax.fori_loop(0, EB // 16, scan_body, jnp.int32(0))

                for j in range(SUB // 16):
                    lane = j * 16 + lax.iota(jnp.int32, 16)
                    csrc[pl.ds(cnt + j * 16, 16)] = lane
                    cdst[pl.ds(cnt + j * 16, 16)] = chunk + lane
                nfill = (nstatic * SUB - cnt - SUB + 15) // 16

                def fill_body(k, car):
                    off = cnt + SUB + k * 16
                    lane = (k % 8) * 16 + lax.iota(jnp.int32, 16)
                    csrc[pl.ds(off, 16)] = lane
                    cdst[pl.ds(off, 16)] = chunk + lane
                    return car
                lax.fori_loop(0, nfill, fill_body, 0)

                drain_scatters()
                for j in range(nstatic):
                    pltpu.async_copy(g_hbm.at[csrc.at[pl.ds(j * SUB, SUB)]],
                                     rows.at[pl.ds(j * SUB, SUB)], sem_g)
                for j in range(nstatic):
                    pltpu.make_async_copy(
                        g_hbm.at[csrc.at[pl.ds(j * SUB, SUB)]],
                        rows.at[pl.ds(j * SUB, SUB)], sem_g).wait()
                for j in range(nstatic):
                    for jj in range(SUB // 16):
                        stage[j, pl.ds(jj * 16, 16)] = (
                            cdst[pl.ds(j * SUB + jj * 16, 16)])
                for j in range(nstatic):
                    pltpu.async_copy(rows.at[pl.ds(j * SUB, SUB)],
                                     acc.at[stage.at[j]], sem_s, add=True)

                nsub = (cnt + SUB - 1) // SUB

                @pl.when(nsub > nstatic)
                def _():
                    drain_scatters()

                    def over_body(sb, car):
                        base = sb * SUB
                        for jj in range(SUB // 16):
                            stage[0, pl.ds(jj * 16, 16)] = (
                                cdst[pl.ds(base + jj * 16, 16)])
                        pltpu.sync_copy(g_hbm.at[csrc.at[pl.ds(base, SUB)]],
                                        rows.at[pl.ds(0, SUB)])
                        pltpu.sync_copy(rows.at[pl.ds(0, SUB)],
                                        acc.at[stage.at[0]], add=True)
                        return car
                    lax.fori_loop(nstatic, nsub, over_body, 0)
                    fire_dummy_scatters()

            fire_dummy_scatters()
            issue_edges(0, 0)
            issue_edges(1, 1)

            @pl.loop(0, NBLK // 2 - 1)
            def _(i):
                b0 = 2 * i
                b1 = 2 * i + 1
                wait_edges(0, b0)
                process(0, b0)
                issue_edges(0, b0 + 2)
                wait_edges(1, b1)
                process(1, b1)
                issue_edges(1, b1 + 2)

            wait_edges(0, NBLK - 2)
            process(0, NBLK - 2)
            wait_edges(1, NBLK - 1)
            process(1, NBLK - 1)
            drain_scatters()

            plsc.subcore_barrier()
            pltpu.sync_copy(acc.at[pl.ds(tid * wb, wb)],
                            agg_hbm.at[pl.ds(lo + tid * wb, wb)])
            plsc.subcore_barrier()

    return agg_kernel


def _make_emb_lookup(v, d):
    mesh = plsc.VectorSubcoreMesh(core_axis_name="c", subcore_axis_name="s")
    per_w = NPAD // (NCORES * NTILES)
    blk = 112
    nblk = per_w // blk

    @functools.partial(
        pl.kernel,
        out_type=jax.ShapeDtypeStruct((NPAD, d), jnp.float32),
        mesh=mesh,
        scratch_types=[
            pltpu.VMEM((blk,), jnp.int32),
            pltpu.VMEM((blk, d), jnp.float32),
        ],
        compiler_params=pltpu.CompilerParams(use_tc_tiling_on_sc=False),
    )
    def emb_kernel(emb_hbm, x_hbm, out_hbm, idx_v, rows_v):
        wid = lax.axis_index("s") * NCORES + lax.axis_index("c")
        base = wid * per_w

        @pl.loop(0, nblk)
        def _(b):
            off = base + b * blk
            pltpu.sync_copy(x_hbm.at[pl.ds(off, blk)], idx_v)
            pltpu.sync_copy(emb_hbm.at[idx_v], rows_v)
            pltpu.sync_copy(rows_v, out_hbm.at[pl.ds(off, blk)])

    return emb_kernel


ROWB = 2048


def _pre_kernel(h_ref, ws_ref, bs_ref, wd_ref, bd_ref, g_ref, hd_ref):
    h = h_ref[...]
    hs = jnp.dot(h, ws_ref[...], preferred_element_type=jnp.float32) + bs_ref[...]
    g_ref[...] = jnp.maximum(hs, 0.0) + EPS
    hd_ref[...] = (jnp.dot(h, wd_ref[...], preferred_element_type=jnp.float32)
                   + bd_ref[...])


def _tc_pre(h, ws, bs, wd, bd):
    din, dout = ws.shape
    grid = NPAD // ROWB
    return pl.pallas_call(
        _pre_kernel,
        grid=(grid,),
        in_specs=[
            pl.BlockSpec((ROWB, din), lambda i: (i, 0)),
            pl.BlockSpec((din, dout), lambda i: (0, 0)),
            pl.BlockSpec((1, dout), lambda i: (0, 0)),
            pl.BlockSpec((din, dout), lambda i: (0, 0)),
            pl.BlockSpec((1, dout), lambda i: (0, 0)),
        ],
        out_specs=[
            pl.BlockSpec((ROWB, dout), lambda i: (i, 0)),
            pl.BlockSpec((ROWB, dout), lambda i: (i, 0)),
        ],
        out_shape=[
            jax.ShapeDtypeStruct((NPAD, dout), jnp.float32),
            jax.ShapeDtypeStruct((NPAD, dout), jnp.float32),
        ],
    )(h, ws, bs.reshape(1, -1), wd, bd.reshape(1, -1))


def _post_kernel(make_g, agg_ref, hd_ref, w_ref, b_ref, *out_refs):
    s = agg_ref[...] + hd_ref[...]
    h = jnp.dot(s, w_ref[...], preferred_element_type=jnp.float32) + b_ref[...]
    out_refs[0][...] = h
    if make_g:
        out_refs[1][...] = jnp.maximum(h, 0.0) + EPS


def _tc_post(agg, hd, w, b, make_g):
    din, dout = w.shape
    grid = NPAD // ROWB
    out_specs = [pl.BlockSpec((ROWB, dout), lambda i: (i, 0))]
    out_shape = [jax.ShapeDtypeStruct((NPAD, dout), jnp.float32)]
    if make_g:
        out_specs.append(pl.BlockSpec((ROWB, dout), lambda i: (i, 0)))
        out_shape.append(jax.ShapeDtypeStruct((NPAD, dout), jnp.float32))
    return pl.pallas_call(
        functools.partial(_post_kernel, make_g),
        grid=(grid,),
        in_specs=[
            pl.BlockSpec((ROWB, din), lambda i: (i, 0)),
            pl.BlockSpec((ROWB, din), lambda i: (i, 0)),
            pl.BlockSpec((din, dout), lambda i: (0, 0)),
            pl.BlockSpec((1, dout), lambda i: (0, 0)),
        ],
        out_specs=out_specs,
        out_shape=out_shape,
    )(agg, hd, w, b.reshape(1, -1))


def _pool_kernel(h_ref, bf_ref, demo_ref, w1a_ref, w1b_ref, b1_ref, w2_ref,
                 b2_ref, out_ref, sums_ref, cnts_ref):
    i = pl.program_id(0)
    n = pl.num_programs(0)

    @pl.when(i == 0)
    def _():
        sums_ref[...] = jnp.zeros_like(sums_ref)
        cnts_ref[...] = jnp.zeros_like(cnts_ref)

    gids = lax.broadcasted_iota(jnp.int32, (1, NUM_GRAPHS), 1)
    onehot = (bf_ref[...] == gids).astype(jnp.float32)
    sums_ref[...] += lax.dot_general(
        onehot, h_ref[...], (((0,), (0,)), ((), ())),
        preferred_element_type=jnp.float32)
    cnts_ref[...] += lax.dot_general(
        onehot, jnp.ones((ROWB, 1), jnp.float32), (((0,), (0,)), ((), ())),
        preferred_element_type=jnp.float32)

    @pl.when(i == n - 1)
    def _():
        gf = sums_ref[...] / jnp.maximum(cnts_ref[...], 1.0)
        z = (jnp.dot(gf, w1a_ref[...], preferred_element_type=jnp.float32)
             + jnp.dot(demo_ref[...], w1b_ref[...],
                       preferred_element_type=jnp.float32)
             + b1_ref[...])
        z = jnp.maximum(z, 0.0)
        out_ref[...] = (jnp.dot(z, w2_ref[...],
                                preferred_element_type=jnp.float32)
                        + b2_ref[...])


def _tc_pool_cls(h, batch_f, demo, w1, b1, w2, b2):
    grid = NPAD // ROWB
    md = w1.shape[1]
    od = w2.shape[1]
    nd = demo.shape[1]
    return pl.pallas_call(
        _pool_kernel,
        grid=(grid,),
        in_specs=[
            pl.BlockSpec((ROWB, h.shape[1]), lambda i: (i, 0)),
            pl.BlockSpec((ROWB, 1), lambda i: (i, 0)),
            pl.BlockSpec((NUM_GRAPHS, nd), lambda i: (0, 0)),
            pl.BlockSpec((NUM_GRAPHS, md), lambda i: (0, 0)),
            pl.BlockSpec((nd, md), lambda i: (0, 0)),
            pl.BlockSpec((1, md), lambda i: (0, 0)),
            pl.BlockSpec((md, od), lambda i: (0, 0)),
            pl.BlockSpec((1, od), lambda i: (0, 0)),
        ],
        out_specs=pl.BlockSpec((NUM_GRAPHS, od), lambda i: (0, 0)),
        out_shape=jax.ShapeDtypeStruct((NUM_GRAPHS, od), jnp.float32),
        scratch_shapes=[
            pltpu.VMEM((NUM_GRAPHS, NUM_GRAPHS), jnp.float32),
            pltpu.VMEM((NUM_GRAPHS, 1), jnp.float32),
        ],
    )(h, batch_f, demo, w1[:NUM_GRAPHS], w1[NUM_GRAPHS:], b1.reshape(1, -1),
      w2, b2.reshape(1, -1))


def kernel(x, edge_index, batch, demographics, emb, l0_src_w, l0_src_b, l0_dst_w,
           l0_dst_b, l0_mlp_w, l0_mlp_b, l1_mlp_w, l1_mlp_b, l2_src_w, l2_src_b,
           l2_dst_w, l2_dst_b, l2_mlp_w, l2_mlp_b, cls_w1, cls_b1, cls_w2, cls_b2):
    pad = NPAD - N_NODES
    x_pad = jnp.concatenate([x.astype(jnp.int32), jnp.zeros((pad,), jnp.int32)])
    batch_f = jnp.concatenate(
        [batch.astype(jnp.int32),
         jnp.full((pad,), NUM_GRAPHS, jnp.int32)]).reshape(NPAD, 1)
    edges = edge_index.astype(jnp.int32)

    agg48 = _make_agg(48, 2, 5)
    agg64 = _make_agg(64, 4, 3)

    h = _make_emb_lookup(emb.shape[0], emb.shape[1])(emb, x_pad)
    g, hd = _tc_pre(h, l0_src_w, l0_src_b, l0_dst_w, l0_dst_b)
    agg = agg48(g, edges)
    h, g = _tc_post(agg, hd, l0_mlp_w, l0_mlp_b, make_g=True)
    agg = agg48(g, edges)
    (h,) = _tc_post(agg, h, l1_mlp_w, l1_mlp_b, make_g=False)
    g, hd = _tc_pre(h, l2_src_w, l2_src_b, l2_dst_w, l2_dst_b)
    agg = agg64(g, edges)
    (h,) = _tc_post(agg, hd, l2_mlp_w, l2_mlp_b, make_g=False)
    return _tc_pool_cls(h, batch_f, demographics, cls_w1, cls_b1, cls_w2, cls_b2)
